# baseline (speedup 1.0000x reference)
import functools

import jax
import jax.numpy as jnp
from jax import lax
from jax.experimental import pallas as pl
from jax.experimental.pallas import tpu as pltpu
from jax.experimental.pallas import tpu_sc as plsc

NO_EMBEDS = 1000000
EMBED_DIM = 64
BATCH = 4096
HIST_LEN = 200

HALF = NO_EMBEDS // 2
CB = 19200
NTB = -(-NO_EMBEDS // CB)

ROWS = BATCH * HIST_LEN
GROUP = 128
NGROUPS = ROWS // GROUP
NWORKERS = 32
GPW = NGROUPS // NWORKERS
K = 5
NCHUNKS = GPW // K

BB = 256
L2B = 25


def _pack_table(table_t):
  def body(x_ref, o_ref):
    x = x_ref[...]
    eye = jnp.eye(EMBED_DIM, dtype=jnp.float32)
    hi = x.astype(jnp.bfloat16)
    lo = (x - hi.astype(jnp.float32)).astype(jnp.bfloat16)
    dims = (((0,), (0,)), ((), ()))
    y = (lax.dot_general(hi, eye.astype(jnp.bfloat16), dims,
                         preferred_element_type=jnp.float32) +
         lax.dot_general(lo, eye.astype(jnp.bfloat16), dims,
                         preferred_element_type=jnp.float32))
    y3 = y.reshape(CB // 2, 2, EMBED_DIM)
    o_ref[...] = jnp.concatenate([y3[:, 0, :], y3[:, 1, :]], axis=1)

  return pl.pallas_call(
      body,
      grid=(NTB,),
      in_specs=[pl.BlockSpec((EMBED_DIM, CB), lambda i: (0, i))],
      out_specs=pl.BlockSpec((CB // 2, 2 * EMBED_DIM), lambda i: (i, 0)),
      out_shape=jax.ShapeDtypeStruct((HALF, 2 * EMBED_DIM), jnp.float32),
  )(table_t)


def _build_gather():
  mesh = plsc.VectorSubcoreMesh(core_axis_name="c", subcore_axis_name="s")

  @functools.partial(
      pl.kernel,
      mesh=mesh,
      out_type=jax.ShapeDtypeStruct((NGROUPS, GROUP, EMBED_DIM), jnp.float32),
      scratch_types=[
          pltpu.VMEM((GPW, GROUP), jnp.int32),
          pltpu.VMEM((2, K, GROUP, EMBED_DIM), jnp.float32),
          pltpu.SemaphoreType.DMA,
          pltpu.SemaphoreType.DMA,
      ],
      compiler_params=pltpu.CompilerParams(use_tc_tiling_on_sc=False),
  )
  def gather_kernel(idx_hbm, table_hbm, out_hbm, idx_v, rows_v, sem0, sem1):
    wid = lax.axis_index("s") * 2 + lax.axis_index("c")
    base = wid * GPW
    sems = (sem0, sem1)

    pltpu.sync_copy(idx_hbm.at[pl.ds(base, GPW)], idx_v)

    def fire(c, slot):
      for j in range(K):
        pltpu.async_copy(
            table_hbm.at[idx_v.at[c * K + j]], rows_v.at[slot].at[j],
            sems[slot])

    def drain_write(c, slot):
      dst = out_hbm.at[pl.ds(base + c * K, K)]
      pltpu.make_async_copy(dst, rows_v.at[slot], sems[slot]).wait()
      pltpu.sync_copy(rows_v.at[slot], dst)

    fire(0, 0)

    def body(i, carry):
      c0 = 2 * i
      fire(c0 + 1, 1)
      drain_write(c0, 0)

      @pl.when(c0 + 2 < NCHUNKS)
      def _():
        fire(c0 + 2, 0)

      drain_write(c0 + 1, 1)
      return carry

    lax.fori_loop(0, NCHUNKS // 2, body, 0)

  return gather_kernel


_gather = _build_gather()


def _unpack_out(gathered):
  def body(x_ref, o_ref):
    eye = jnp.eye(EMBED_DIM, dtype=jnp.float32)
    for p in range(L2B):
      x = x_ref[:, pl.ds(p * 2 * EMBED_DIM, 2 * EMBED_DIM)]
      o_ref[2 * p] = lax.dot_general(eye, x[:, 0:EMBED_DIM],
                                     (((1,), (1,)), ((), ())),
                                     preferred_element_type=jnp.float32)
      o_ref[2 * p + 1] = lax.dot_general(eye, x[:, EMBED_DIM:2 * EMBED_DIM],
                                         (((1,), (1,)), ((), ())),
                                         preferred_element_type=jnp.float32)

  return pl.pallas_call(
      body,
      grid=(BATCH // BB, HIST_LEN // (2 * L2B)),
      in_specs=[
          pl.BlockSpec((BB, 2 * EMBED_DIM * L2B), lambda bi, l2: (bi, l2))
      ],
      out_specs=pl.BlockSpec((2 * L2B, EMBED_DIM, BB),
                             lambda bi, l2: (l2, 0, bi)),
      out_shape=jax.ShapeDtypeStruct((HIST_LEN, EMBED_DIM, BATCH), jnp.float32),
  )(gathered)


def kernel(language_f, narration_embeds):
  table_t = narration_embeds.T
  packed = _pack_table(table_t)
  gathered = _gather(language_f.reshape(NGROUPS, GROUP),
                     packed.reshape(NO_EMBEDS, EMBED_DIM))
  out_t = _unpack_out(gathered.reshape(BATCH, HIST_LEN * EMBED_DIM))
  return out_t.transpose(2, 0, 1)


# speedup vs baseline: 1.1617x; 1.1617x over previous
import functools

import jax
import jax.numpy as jnp
from jax import lax
from jax.experimental import pallas as pl
from jax.experimental.pallas import tpu as pltpu
from jax.experimental.pallas import tpu_sc as plsc

NO_EMBEDS = 1000000
EMBED_DIM = 64
BATCH = 4096
HIST_LEN = 200

HALF = NO_EMBEDS // 2
CB = 19200
NTB = -(-NO_EMBEDS // CB)

HC = 4
LC = HIST_LEN // HC
GROUP = 128
NG_H = BATCH * LC // GROUP
NWORKERS = 32
GPW = NG_H // NWORKERS
K = 5
NCHUNKS = GPW // K

BB = 128


def _pack_table(table_t):
  def body(x_ref, o_ref):
    y = x_ref[...].T
    y3 = y.reshape(CB // 2, 2, EMBED_DIM)
    o_ref[...] = jnp.concatenate([y3[:, 0, :], y3[:, 1, :]], axis=1)

  return pl.pallas_call(
      body,
      grid=(NTB,),
      in_specs=[pl.BlockSpec((EMBED_DIM, CB), lambda i: (0, i))],
      out_specs=pl.BlockSpec((CB // 2, 2 * EMBED_DIM), lambda i: (i, 0)),
      out_shape=jax.ShapeDtypeStruct((HALF, 2 * EMBED_DIM), jnp.float32),
  )(table_t)


def _build_gather():
  mesh = plsc.VectorSubcoreMesh(core_axis_name="c", subcore_axis_name="s")

  @functools.partial(
      pl.kernel,
      mesh=mesh,
      out_type=jax.ShapeDtypeStruct((NG_H, GROUP, EMBED_DIM), jnp.float32),
      scratch_types=[
          pltpu.VMEM((GPW, GROUP), jnp.int32),
          pltpu.VMEM((2, K, GROUP, EMBED_DIM), jnp.float32),
          pltpu.SemaphoreType.DMA,
          pltpu.SemaphoreType.DMA,
      ],
      compiler_params=pltpu.CompilerParams(use_tc_tiling_on_sc=False),
  )
  def gather_kernel(idx_hbm, table_hbm, out_hbm, idx_v, rows_v, sem0, sem1):
    wid = lax.axis_index("s") * 2 + lax.axis_index("c")
    base = wid * GPW
    sems = (sem0, sem1)

    pltpu.sync_copy(idx_hbm.at[pl.ds(base, GPW)], idx_v)

    def fire(c, slot):
      for j in range(K):
        pltpu.async_copy(
            table_hbm.at[idx_v.at[c * K + j]], rows_v.at[slot].at[j],
            sems[slot])

    def drain_write(c, slot):
      dst = out_hbm.at[pl.ds(base + c * K, K)]
      pltpu.make_async_copy(dst, rows_v.at[slot], sems[slot]).wait()
      pltpu.sync_copy(rows_v.at[slot], dst)

    fire(0, 0)

    def body(i, carry):
      c0 = 2 * i
      fire(c0 + 1, 1)
      drain_write(c0, 0)

      @pl.when(c0 + 2 < NCHUNKS)
      def _():
        fire(c0 + 2, 0)

      drain_write(c0 + 1, 1)
      return carry

    lax.fori_loop(0, NCHUNKS // 2, body, 0)

  return gather_kernel


_gather = _build_gather()


def _unpack_out(g_views):
  def body(x0_ref, x1_ref, x2_ref, x3_ref, o_ref):
    eye = jnp.eye(EMBED_DIM, dtype=jnp.float32)
    for h, x_ref in enumerate((x0_ref, x1_ref, x2_ref, x3_ref)):
      for l2 in range(LC // 2):
        x = x_ref[:, pl.ds(l2 * 2 * EMBED_DIM, 2 * EMBED_DIM)]
        l = LC * h + 2 * l2
        o_ref[l] = lax.dot_general(eye, x[:, 0:EMBED_DIM],
                                   (((1,), (1,)), ((), ())),
                                   preferred_element_type=jnp.float32)
        o_ref[l + 1] = lax.dot_general(eye, x[:, EMBED_DIM:2 * EMBED_DIM],
                                       (((1,), (1,)), ((), ())),
                                       preferred_element_type=jnp.float32)

  spec = pl.BlockSpec((BB, LC * EMBED_DIM), lambda bi: (bi, 0))
  return pl.pallas_call(
      body,
      grid=(BATCH // BB,),
      in_specs=[spec, spec, spec, spec],
      out_specs=pl.BlockSpec((HIST_LEN, EMBED_DIM, BB), lambda bi: (0, 0, bi)),
      out_shape=jax.ShapeDtypeStruct((HIST_LEN, EMBED_DIM, BATCH), jnp.float32),
  )(*g_views)


def kernel(language_f, narration_embeds):
  table_t = narration_embeds.T
  packed = _pack_table(table_t)
  table_v = packed.reshape(NO_EMBEDS, EMBED_DIM)
  g_views = []
  for h in range(HC):
    idx_h = lax.slice(language_f, (0, h * LC), (BATCH, (h + 1) * LC))
    g = _gather(idx_h.reshape(NG_H, GROUP), table_v)
    g_views.append(g.reshape(BATCH, LC * EMBED_DIM))
  out_t = _unpack_out(g_views)
  return out_t.transpose(2, 0, 1)
